# baseline (speedup 1.0000x reference)
import functools

import jax
import jax.numpy as jnp
from jax import lax
from jax.experimental import pallas as pl
from jax.experimental.pallas import tpu as pltpu
from jax.experimental.pallas import tpu_sc as plsc

N_NODES = 10000
N_EDGES = 320000
D_FEAT = 128
D_EDGE = 16

_NC = 2
_NS = 16
_NW = _NC * _NS
_PER_W = N_EDGES // _NW
_CH = 80
_ROWS_W = _PER_W // _CH
_NPART = 5
_PR = _ROWS_W // _NPART
_SA = 9
_SB = 8
_CP = _CH // 2
_GPW = _PR * _CP
_EPW = _PR * _CH


def _tables_body(x_ref, ws_ref, wd_ref, b_ref, xs_ref, xd_ref):
    x = x_ref[...]
    xs_ref[...] = (
        jnp.dot(x, ws_ref[...], preferred_element_type=jnp.float32) + b_ref[...]
    )
    xd_ref[...] = jnp.dot(x, wd_ref[...], preferred_element_type=jnp.float32)


def _node_tables(x, ws, wd, b2d):
    return pl.pallas_call(
        _tables_body,
        out_shape=(
            jax.ShapeDtypeStruct((N_NODES, D_FEAT), jnp.float32),
            jax.ShapeDtypeStruct((N_NODES, D_FEAT), jnp.float32),
        ),
    )(x, ws, wd, b2d)


def _make_gather_add_body(part):
    def _gather_add_body(
        xs_hbm, xd_hbm, src2_hbm, dst2_hbm, out_hbm,
        idx_s, idx_d,
        rsa, rda, oa, rsb, rdb, ob, rsc, rdc, oc,
        gsem_a, gsem_b, gsem_c,
        ssem_a, ssem_b, ssem_c,
    ):
        wid = lax.axis_index("s") * _NC + lax.axis_index("c")

        pltpu.sync_copy(src2_hbm.at[wid, part], idx_s)
        pltpu.sync_copy(dst2_hbm.at[wid, part], idx_d)

        def issue(r, rs, rd, gsem):
            pltpu.async_copy(xs_hbm.at[idx_s.at[r]], rs, gsem)
            pltpu.async_copy(xd_hbm.at[idx_d.at[r]], rd, gsem)

        def wait_gathers(rs, rd, gsem):
            pltpu.make_async_copy(xs_hbm.at[pl.ds(0, _CH)], rs, gsem).wait()
            pltpu.make_async_copy(xd_hbm.at[pl.ds(0, _CH)], rd, gsem).wait()

        def drain_store(o, ssem):
            pltpu.make_async_copy(o, out_hbm.at[0, pl.ds(0, _CP)], ssem).wait()

        u32 = jnp.uint32
        f32 = jnp.float32

        def rne_pack(a, b):
            ua = plsc.bitcast(a, u32)
            ub = plsc.bitcast(b, u32)
            blo = (ua + u32(0x7FFF) + ((ua >> 16) & u32(1))) >> 16
            bhi = (ub + u32(0x7FFF) + ((ub >> 16) & u32(1))) & u32(0xFFFF0000)
            return plsc.bitcast(blo | bhi, f32)

        def add_store(r, rs, rd, o, ssem):
            @plsc.parallel_loop(0, _CP, unroll=2)
            def _(q):
                for e in range(2):
                    row = 2 * q + e
                    for j in range(D_FEAT // 32):
                        sl = pl.ds(j * 16, 16)
                        sh = pl.ds(64 + j * 16, 16)
                        a = rs[row, sl] + rd[row, sl]
                        b = rs[row, sh] + rd[row, sh]
                        o[q, pl.ds(e * 64 + j * 16, 16)] = rne_pack(a, b)

            off = pl.multiple_of(r * _CP, 8)
            pltpu.async_copy(o, out_hbm.at[wid, pl.ds(off, _CP)], ssem)

        streams = (
            (0, _SA, rsa, rda, oa, gsem_a, ssem_a),
            (_SA, _SB, rsb, rdb, ob, gsem_b, ssem_b),
            (_SA + _SB, _SB, rsc, rdc, oc, gsem_c, ssem_c),
        )

        for start, _, rs, rd, _, gsem, _ in streams:
            issue(start, rs, rd, gsem)

        def body(i, carry):
            for start, size, rs, rd, o, gsem, ssem in streams:
                wait_gathers(rs, rd, gsem)

                @pl.when(i > 0)
                def _():
                    drain_store(o, ssem)

                add_store(start + i, rs, rd, o, ssem)
                issue(jnp.minimum(start + i + 1, start + size - 1), rs, rd, gsem)
            return carry

        lax.fori_loop(0, _SB, body, 0)

        wait_gathers(rsa, rda, gsem_a)
        drain_store(oa, ssem_a)
        add_store(_SA - 1, rsa, rda, oa, ssem_a)
        drain_store(oa, ssem_a)
        for _, _, rs, rd, o, gsem, ssem in streams[1:]:
            wait_gathers(rs, rd, gsem)
            drain_store(o, ssem)

    return _gather_add_body


def _gather_add_part(xs, xd, src2, dst2, part):
    mesh = plsc.VectorSubcoreMesh(core_axis_name="c", subcore_axis_name="s")
    fn = functools.partial(
        pl.kernel,
        out_type=jax.ShapeDtypeStruct((_NW, _GPW, D_FEAT), jnp.float32),
        mesh=mesh,
        compiler_params=pltpu.CompilerParams(needs_layout_passes=False),
        scratch_types=[
            pltpu.VMEM((_PR, _CH), jnp.int32),
            pltpu.VMEM((_PR, _CH), jnp.int32),
            pltpu.VMEM((_CH, D_FEAT), jnp.float32),
            pltpu.VMEM((_CH, D_FEAT), jnp.float32),
            pltpu.VMEM((_CP, D_FEAT), jnp.float32),
            pltpu.VMEM((_CH, D_FEAT), jnp.float32),
            pltpu.VMEM((_CH, D_FEAT), jnp.float32),
            pltpu.VMEM((_CP, D_FEAT), jnp.float32),
            pltpu.VMEM((_CH, D_FEAT), jnp.float32),
            pltpu.VMEM((_CH, D_FEAT), jnp.float32),
            pltpu.VMEM((_CP, D_FEAT), jnp.float32),
            pltpu.SemaphoreType.DMA,
            pltpu.SemaphoreType.DMA,
            pltpu.SemaphoreType.DMA,
            pltpu.SemaphoreType.DMA,
            pltpu.SemaphoreType.DMA,
            pltpu.SemaphoreType.DMA,
        ],
    )(_make_gather_add_body(part))
    return fn(xs, xd, src2, dst2)


def _final_body(g_ref, ea_ref, we_ref, *rest):
    o_ref = rest[-1]
    acc = jnp.dot(ea_ref[...], we_ref[...], preferred_element_type=jnp.float32)
    w = lax.bitcast_convert_type(g_ref[...].reshape(_GPW, D_FEAT), jnp.uint32)
    lo = lax.bitcast_convert_type(w << 16, jnp.float32)
    hi = lax.bitcast_convert_type(w & jnp.uint32(0xFFFF0000), jnp.float32)
    e0 = jnp.concatenate([lo[:, :64], hi[:, :64]], axis=1)
    e1 = jnp.concatenate([lo[:, 64:], hi[:, 64:]], axis=1)
    g = jnp.stack([e0, e1], axis=1).reshape(_EPW, D_FEAT)
    o_ref[...] = jnp.maximum(acc + g, 0.0)


def _final_part(g, edge_attr, we, out_prev, part):
    in_specs = [
        pl.BlockSpec((1, _GPW, D_FEAT), lambda i: (i, 0, 0)),
        pl.BlockSpec((_EPW, D_EDGE), lambda i, _p=part: (i * _NPART + _p, 0)),
        pl.BlockSpec((D_EDGE, D_FEAT), lambda i: (0, 0)),
    ]
    args = [g, edge_attr, we]
    aliases = {}
    if out_prev is not None:
        in_specs.append(pl.BlockSpec(memory_space=pl.ANY))
        args.append(out_prev)
        aliases = {3: 0}
    return pl.pallas_call(
        _final_body,
        grid=(_NW,),
        in_specs=in_specs,
        out_specs=pl.BlockSpec(
            (_EPW, D_FEAT), lambda i, _p=part: (i * _NPART + _p, 0)
        ),
        out_shape=jax.ShapeDtypeStruct((N_EDGES, D_FEAT), jnp.float32),
        input_output_aliases=aliases,
    )(*args)


def kernel(x, edge_index, edge_attr, W, b):
    we = W[:D_EDGE]
    ws = W[D_EDGE : D_EDGE + D_FEAT]
    wd = W[D_EDGE + D_FEAT :]
    b2d = b.reshape(1, D_FEAT)
    src2 = edge_index[0].reshape(_NW, _NPART, _PR, _CH)
    dst2 = edge_index[1].reshape(_NW, _NPART, _PR, _CH)

    xs, xd = _node_tables(x, ws, wd, b2d)
    gs = [_gather_add_part(xs, xd, src2, dst2, k) for k in range(_NPART)]
    out = _final_part(gs[0], edge_attr, we, None, 0)
    for k in range(1, _NPART):
        out = _final_part(gs[k], edge_attr, we, out, k)
    return out


# speedup vs baseline: 1.0493x; 1.0493x over previous
import functools

import jax
import jax.numpy as jnp
from jax import lax
from jax.experimental import pallas as pl
from jax.experimental.pallas import tpu as pltpu
from jax.experimental.pallas import tpu_sc as plsc

N_NODES = 10000
N_EDGES = 320000
D_FEAT = 128
D_EDGE = 16

_NC = 2
_NS = 16
_NW = _NC * _NS
_PER_W = N_EDGES // _NW
_CH = 80
_ROWS_W = _PER_W // _CH
_SA = 43
_SB = 41
_CP = _CH // 2
_NP = N_EDGES // 2


def _tables_body(x_ref, ws_ref, wd_ref, b_ref, xs_ref, xd_ref):
    x = x_ref[...]
    xs_ref[...] = (
        jnp.dot(x, ws_ref[...], preferred_element_type=jnp.float32) + b_ref[...]
    )
    xd_ref[...] = jnp.dot(x, wd_ref[...], preferred_element_type=jnp.float32)


def _node_tables(x, ws, wd, b2d):
    return pl.pallas_call(
        _tables_body,
        out_shape=(
            jax.ShapeDtypeStruct((N_NODES, D_FEAT), jnp.float32),
            jax.ShapeDtypeStruct((N_NODES, D_FEAT), jnp.float32),
        ),
    )(x, ws, wd, b2d)


def _gather_add_body(
    xs_hbm, xd_hbm, src2_hbm, dst2_hbm, out_hbm,
    idx_s, idx_d,
    rsa, rda, oa, rsb, rdb, ob, rsc, rdc, oc,
    gsem_a, gsem_b, gsem_c,
    ssem_a, ssem_b, ssem_c,
):
    wid = lax.axis_index("s") * _NC + lax.axis_index("c")
    ebase = wid * _PER_W

    pltpu.sync_copy(src2_hbm.at[wid], idx_s)
    pltpu.sync_copy(dst2_hbm.at[wid], idx_d)

    def issue(r, rs, rd, gsem):
        pltpu.async_copy(xs_hbm.at[idx_s.at[r]], rs, gsem)
        pltpu.async_copy(xd_hbm.at[idx_d.at[r]], rd, gsem)

    def wait_gathers(rs, rd, gsem):
        pltpu.make_async_copy(xs_hbm.at[pl.ds(0, _CH)], rs, gsem).wait()
        pltpu.make_async_copy(xd_hbm.at[pl.ds(0, _CH)], rd, gsem).wait()

    def drain_store(o, ssem):
        pltpu.make_async_copy(o, out_hbm.at[pl.ds(0, _CP)], ssem).wait()

    u32 = jnp.uint32
    f32 = jnp.float32

    def rne_pack(a, b):
        ua = plsc.bitcast(a, u32)
        ub = plsc.bitcast(b, u32)
        blo = (ua + u32(0x7FFF) + ((ua >> 16) & u32(1))) >> 16
        bhi = (ub + u32(0x7FFF) + ((ub >> 16) & u32(1))) & u32(0xFFFF0000)
        return plsc.bitcast(blo | bhi, f32)

    def add_store(r, rs, rd, o, ssem):
        @plsc.parallel_loop(0, _CP, unroll=2)
        def _(q):
            for e in range(2):
                row = 2 * q + e
                for j in range(D_FEAT // 32):
                    sl = pl.ds(j * 16, 16)
                    sh = pl.ds(64 + j * 16, 16)
                    a = rs[row, sl] + rd[row, sl]
                    b = rs[row, sh] + rd[row, sh]
                    o[q, pl.ds(e * 64 + j * 16, 16)] = rne_pack(a, b)

        off = pl.multiple_of((ebase + r * _CH) // 2, 8)
        pltpu.async_copy(o, out_hbm.at[pl.ds(off, _CP)], ssem)

    streams = (
        (0, _SA, rsa, rda, oa, gsem_a, ssem_a),
        (_SA, _SB, rsb, rdb, ob, gsem_b, ssem_b),
        (_SA + _SB, _SB, rsc, rdc, oc, gsem_c, ssem_c),
    )

    for start, _, rs, rd, _, gsem, _ in streams:
        issue(start, rs, rd, gsem)

    def body(i, carry):
        for start, size, rs, rd, o, gsem, ssem in streams:
            wait_gathers(rs, rd, gsem)

            @pl.when(i > 0)
            def _():
                drain_store(o, ssem)

            add_store(start + i, rs, rd, o, ssem)
            issue(jnp.minimum(start + i + 1, start + size - 1), rs, rd, gsem)
        return carry

    lax.fori_loop(0, _SB, body, 0)

    wait_gathers(rsa, rda, gsem_a)
    drain_store(oa, ssem_a)
    add_store(_SB, rsa, rda, oa, ssem_a)
    issue(_SA - 1, rsa, rda, gsem_a)
    wait_gathers(rsa, rda, gsem_a)
    drain_store(oa, ssem_a)
    add_store(_SA - 1, rsa, rda, oa, ssem_a)
    drain_store(oa, ssem_a)
    for _, _, rs, rd, o, gsem, ssem in streams[1:]:
        wait_gathers(rs, rd, gsem)
        drain_store(o, ssem)


def _gather_add(xs, xd, src2, dst2):
    mesh = plsc.VectorSubcoreMesh(core_axis_name="c", subcore_axis_name="s")
    fn = functools.partial(
        pl.kernel,
        out_type=jax.ShapeDtypeStruct((_NP, D_FEAT), jnp.float32),
        mesh=mesh,
        compiler_params=pltpu.CompilerParams(needs_layout_passes=False),
        scratch_types=[
            pltpu.VMEM((_ROWS_W, _CH), jnp.int32),
            pltpu.VMEM((_ROWS_W, _CH), jnp.int32),
            pltpu.VMEM((_CH, D_FEAT), jnp.float32),
            pltpu.VMEM((_CH, D_FEAT), jnp.float32),
            pltpu.VMEM((_CP, D_FEAT), jnp.float32),
            pltpu.VMEM((_CH, D_FEAT), jnp.float32),
            pltpu.VMEM((_CH, D_FEAT), jnp.float32),
            pltpu.VMEM((_CP, D_FEAT), jnp.float32),
            pltpu.VMEM((_CH, D_FEAT), jnp.float32),
            pltpu.VMEM((_CH, D_FEAT), jnp.float32),
            pltpu.VMEM((_CP, D_FEAT), jnp.float32),
            pltpu.SemaphoreType.DMA,
            pltpu.SemaphoreType.DMA,
            pltpu.SemaphoreType.DMA,
            pltpu.SemaphoreType.DMA,
            pltpu.SemaphoreType.DMA,
            pltpu.SemaphoreType.DMA,
        ],
    )(_gather_add_body)
    return fn(xs, xd, src2, dst2)


_EBLK = 6400
_HBLK = _EBLK // 2


def _final_body(g_ref, ea_ref, we_ref, o_ref):
    acc = jnp.dot(ea_ref[...], we_ref[...], preferred_element_type=jnp.float32)
    w = lax.bitcast_convert_type(g_ref[...], jnp.uint32)
    lo = lax.bitcast_convert_type(w << 16, jnp.float32)
    hi = lax.bitcast_convert_type(w & jnp.uint32(0xFFFF0000), jnp.float32)
    e0 = jnp.concatenate([lo[:, :64], hi[:, :64]], axis=1)
    e1 = jnp.concatenate([lo[:, 64:], hi[:, 64:]], axis=1)
    g = jnp.stack([e0, e1], axis=1).reshape(_EBLK, D_FEAT)
    o_ref[...] = jnp.maximum(acc + g, 0.0)


def _final(g, edge_attr, we):
    grid = (N_EDGES // _EBLK,)
    return pl.pallas_call(
        _final_body,
        grid=grid,
        in_specs=[
            pl.BlockSpec((_HBLK, D_FEAT), lambda i: (i, 0)),
            pl.BlockSpec((_EBLK, D_EDGE), lambda i: (i, 0)),
            pl.BlockSpec((D_EDGE, D_FEAT), lambda i: (0, 0)),
        ],
        out_specs=pl.BlockSpec((_EBLK, D_FEAT), lambda i: (i, 0)),
        out_shape=jax.ShapeDtypeStruct((N_EDGES, D_FEAT), jnp.float32),
    )(g, edge_attr, we)


def kernel(x, edge_index, edge_attr, W, b):
    we = W[:D_EDGE]
    ws = W[D_EDGE : D_EDGE + D_FEAT]
    wd = W[D_EDGE + D_FEAT :]
    b2d = b.reshape(1, D_FEAT)
    src2 = edge_index[0].reshape(_NW, _ROWS_W, _CH)
    dst2 = edge_index[1].reshape(_NW, _ROWS_W, _CH)

    xs, xd = _node_tables(x, ws, wd, b2d)
    g = _gather_add(xs, xd, src2, dst2)
    return _final(g, edge_attr, we)
